# baseline (speedup 1.0000x reference)
import jax
import jax.numpy as jnp
from jax.experimental import pallas as pl
from jax.experimental.pallas import tpu as pltpu

INPUT_DIM = 1024
LATENT_DIM = 4096
K = 64
TM = 256
NCHUNK = 64
WINDOW = 1 << 16

INT_MIN = -(2**31)


def _count_ge(key, t):
    return jnp.sum((key >= t).astype(jnp.int32), axis=1, keepdims=True)


def _body(x_ref, we_ref, be_ref, wd_ref, bd_ref, sp_ref, rec_ref):
    lat = jax.lax.dot_general(
        x_ref[...], we_ref[...], (((1,), (1,)), ((), ())),
        preferred_element_type=jnp.float32,
    ) + be_ref[...]

    ikey = jax.lax.bitcast_convert_type(lat, jnp.int32)
    key = jnp.where(ikey < 0, ikey ^ jnp.int32(0x7FFFFFFF), ikey)

    cmax = jnp.max(key.reshape(TM, NCHUNK, LATENT_DIM // NCHUNK), axis=2)
    lo = jnp.min(cmax, axis=1, keepdims=True)
    hi = jnp.max(cmax, axis=1, keepdims=True) + 1

    def bis_cond(state):
        lo, hi = state
        return jnp.max(hi - lo) > WINDOW

    def bis_step(state):
        lo, hi = state
        mid = lo + jax.lax.shift_right_logical(hi - lo, 1)
        cnt = _count_ge(key, mid)
        big = cnt >= K
        return jnp.where(big, mid, lo), jnp.where(big, hi, mid)

    lo, hi = jax.lax.while_loop(bis_cond, bis_step, (lo, hi))

    r = K - _count_ge(key, hi)

    def ext_cond(state):
        r, _, _ = state
        return jnp.max(r) > 0

    def ext_step(state):
        r, cap, t = state
        inwin = jnp.logical_and(key >= lo, key < cap)
        m = jnp.max(jnp.where(inwin, key, INT_MIN), axis=1, keepdims=True)
        c = jnp.sum((key == m).astype(jnp.int32), axis=1, keepdims=True)
        live = r > 0
        t = jnp.where(live, m, t)
        cap = jnp.where(live, m, cap)
        return r - jnp.where(live, c, 0), cap, t

    r, _, t = jax.lax.while_loop(ext_cond, ext_step, (r, hi, lo))

    sparse = jnp.where(key >= t, lat, 0.0)
    sp_ref[...] = sparse

    rec = jax.lax.dot_general(
        sparse.astype(jnp.bfloat16), wd_ref[...], (((1,), (1,)), ((), ())),
        preferred_element_type=jnp.float32,
    ) + bd_ref[...]
    rec_ref[...] = rec


@jax.jit
def kernel(x, W_enc, b_enc, W_dec, b_dec):
    B = x.shape[0]
    grid = (B // TM,)
    out = pl.pallas_call(
        _body,
        grid=grid,
        in_specs=[
            pl.BlockSpec((TM, INPUT_DIM), lambda i: (i, 0)),
            pl.BlockSpec((LATENT_DIM, INPUT_DIM), lambda i: (0, 0)),
            pl.BlockSpec((1, LATENT_DIM), lambda i: (0, 0)),
            pl.BlockSpec((INPUT_DIM, LATENT_DIM), lambda i: (0, 0)),
            pl.BlockSpec((1, INPUT_DIM), lambda i: (0, 0)),
        ],
        out_specs=[
            pl.BlockSpec((TM, LATENT_DIM), lambda i: (i, 0)),
            pl.BlockSpec((TM, INPUT_DIM), lambda i: (i, 0)),
        ],
        out_shape=[
            jax.ShapeDtypeStruct((B, LATENT_DIM), jnp.float32),
            jax.ShapeDtypeStruct((B, INPUT_DIM), jnp.float32),
        ],
        compiler_params=pltpu.CompilerParams(
            vmem_limit_bytes=100 * 1024 * 1024,
        ),
    )(x, W_enc, b_enc.reshape(1, LATENT_DIM),
      W_dec.astype(jnp.bfloat16), b_dec.reshape(1, INPUT_DIM))
    sparse, recon = out
    return (recon, sparse)
